# baseline (speedup 1.0000x reference)
import jax
import jax.numpy as jnp
from jax.experimental import pallas as pl
from jax.experimental.pallas import tpu as pltpu


def _round_up(x, m):
    return ((x + m - 1) // m) * m


def _cdiv(a, b):
    return (a + b - 1) // b


def _mod_linear_kernel(x_ref, theta_ref, gamma_ref, bias_ref, out_ref):
    xs = (x_ref[...] * theta_ref[...]).astype(jnp.bfloat16)
    acc = jax.lax.dot_general(xs, gamma_ref[...], (((1,), (1,)), ((), ())),
                              preferred_element_type=jnp.float32)
    out_ref[...] = (acc + bias_ref[...]).astype(out_ref.dtype)


def kernel(x, theta, gamma, bias):
    B, F = x.shape
    T, F2 = gamma.shape
    assert F == F2 and theta.shape == (F,) and bias.shape == (T,)
    dtype = x.dtype

    F_pad = _round_up(F, 128)
    T_pad = _round_up(T, 128)

    tm = min(2048, _round_up(B, 8))
    nb = _cdiv(B, tm)
    B_pad = nb * tm

    x_p = jnp.pad(x, ((0, B_pad - B), (0, F_pad - F)))
    gamma_bf = jnp.pad(gamma, ((0, T_pad - T), (0, F_pad - F))).astype(jnp.bfloat16)
    theta_p = jnp.pad(theta, (0, F_pad - F)).reshape(1, F_pad)
    bias_p = jnp.pad(bias, (0, T_pad - T)).reshape(1, T_pad)

    out = pl.pallas_call(
        _mod_linear_kernel,
        out_shape=jax.ShapeDtypeStruct((B_pad, T_pad), dtype),
        grid=(nb,),
        in_specs=[
            pl.BlockSpec((tm, F_pad), lambda i: (i, 0)),
            pl.BlockSpec((1, F_pad), lambda i: (0, 0)),
            pl.BlockSpec((T_pad, F_pad), lambda i: (0, 0)),
            pl.BlockSpec((1, T_pad), lambda i: (0, 0)),
        ],
        out_specs=pl.BlockSpec((tm, T_pad), lambda i: (i, 0),
                               pipeline_mode=pl.Buffered(buffer_count=1)),
        compiler_params=pltpu.CompilerParams(
            dimension_semantics=("parallel",),
            vmem_limit_bytes=60 * 1024 * 1024,
        ),
    )(x_p, theta_p, gamma_bf, bias_p)

    return out[:B, :T]


# speedup vs baseline: 1.3056x; 1.3056x over previous
import jax
import jax.numpy as jnp
from jax.experimental import pallas as pl
from jax.experimental.pallas import tpu as pltpu


def _round_up(x, m):
    return ((x + m - 1) // m) * m


def _cdiv(a, b):
    return (a + b - 1) // b


def _mod_linear_kernel(x_ref, theta_ref, gamma_ref, bias_ref, out_ref):
    xs = (x_ref[...] * theta_ref[...]).astype(jnp.bfloat16)
    g_bf = gamma_ref[...].astype(jnp.bfloat16)
    acc = jax.lax.dot_general(xs, g_bf, (((1,), (1,)), ((), ())),
                              preferred_element_type=jnp.float32)
    out_ref[...] = (acc + bias_ref[...]).astype(out_ref.dtype)


def kernel(x, theta, gamma, bias):
    B, F = x.shape
    T, F2 = gamma.shape
    assert F == F2 and theta.shape == (F,) and bias.shape == (T,)
    dtype = x.dtype

    F_pad = _round_up(F, 128)
    T_pad = _round_up(T, 128)

    tm = min(1024, _round_up(B, 8))
    nc = 2 if B > tm else 1
    ns = _cdiv(B, tm * nc)
    B_pad = nc * ns * tm

    x_p = jnp.pad(x, ((0, B_pad - B), (0, F_pad - F)))
    gamma_p = jnp.pad(gamma, ((0, T_pad - T), (0, F_pad - F)))
    theta_p = jnp.pad(theta, (0, F_pad - F)).reshape(1, F_pad)
    bias_p = jnp.pad(bias, (0, T_pad - T)).reshape(1, T_pad)

    out = pl.pallas_call(
        _mod_linear_kernel,
        out_shape=jax.ShapeDtypeStruct((B_pad, T_pad), dtype),
        grid=(nc, ns),
        in_specs=[
            pl.BlockSpec((tm, F_pad), lambda c, s: (c * ns + s, 0)),
            pl.BlockSpec((1, F_pad), lambda c, s: (0, 0)),
            pl.BlockSpec((T_pad, F_pad), lambda c, s: (0, 0)),
            pl.BlockSpec((1, T_pad), lambda c, s: (0, 0)),
        ],
        out_specs=pl.BlockSpec((tm, T_pad), lambda c, s: (c * ns + s, 0)),
        compiler_params=pltpu.CompilerParams(
            dimension_semantics=("parallel", "arbitrary"),
            vmem_limit_bytes=48 * 1024 * 1024,
        ),
    )(x_p, theta_p, gamma_p, bias_p)

    return out[:B, :T]
